# baseline (speedup 1.0000x reference)
import functools

import jax
import jax.numpy as jnp
from jax.experimental import pallas as pl
from jax.experimental.pallas import tpu as pltpu

EMBED = 256
N = 8192
BM = 512
BN = 512
GI = N // BM
GJ = N // BN


def _dist_argmin_body(z_ref, w_ref, z2_ref, w2_ref, idx_ref, rmin, ridx):
    j = pl.program_id(1)

    @pl.when(j == 0)
    def _init():
        rmin[...] = jnp.full((BM, 1), jnp.inf, jnp.float32)
        ridx[...] = jnp.zeros((BM, 1), jnp.int32)

    zb = z_ref[...]
    wb = w_ref[...]
    mm = jax.lax.dot_general(
        zb, wb, (((1,), (1,)), ((), ())),
        preferred_element_type=jnp.float32)
    d = (z2_ref[...] - 2.0 * mm) + w2_ref[...]

    bmin = jnp.min(d, axis=1, keepdims=True)
    col = jax.lax.broadcasted_iota(jnp.int32, (BM, BN), 1) + j * BN
    bidx = jnp.min(jnp.where(d == bmin, col, jnp.int32(2**30)),
                   axis=1, keepdims=True)

    better = bmin < rmin[...]
    rmin[...] = jnp.where(better, bmin, rmin[...])
    ridx[...] = jnp.where(better, bidx, ridx[...])

    @pl.when(j == GJ - 1)
    def _flush():
        idx_ref[...] = ridx[...]


@functools.partial(jax.jit)
def _encode(z, W, z2, w2):
    return pl.pallas_call(
        _dist_argmin_body,
        grid=(GI, GJ),
        in_specs=[
            pl.BlockSpec((BM, EMBED), lambda i, j: (i, 0)),
            pl.BlockSpec((BN, EMBED), lambda i, j: (j, 0)),
            pl.BlockSpec((BM, 1), lambda i, j: (i, 0)),
            pl.BlockSpec((1, BN), lambda i, j: (0, j)),
        ],
        out_specs=pl.BlockSpec((BM, 1), lambda i, j: (i, 0)),
        out_shape=jax.ShapeDtypeStruct((N, 1), jnp.int32),
        scratch_shapes=[
            pltpu.VMEM((BM, 1), jnp.float32),
            pltpu.VMEM((BM, 1), jnp.int32),
        ],
        compiler_params=pltpu.CompilerParams(
            dimension_semantics=("parallel", "arbitrary")),
    )(z, W, z2, w2)


def kernel(z, W):
    z2 = jnp.sum(z ** 2, axis=1, keepdims=True)
    w2 = jnp.sum(W ** 2, axis=1)[None, :]
    idx = _encode(z, W, z2, w2)[:, 0]
    z_q = W[idx]
    commitment_loss = jnp.mean((jax.lax.stop_gradient(z_q) - z) ** 2)
    codebook_loss = jnp.mean((z_q - jax.lax.stop_gradient(z)) ** 2)
    vq_loss = codebook_loss + 0.25 * commitment_loss
    z_q_st = z + jax.lax.stop_gradient(z_q - z)
    return (z_q_st, vq_loss)


# speedup vs baseline: 1.1342x; 1.1342x over previous
import functools

import jax
import jax.numpy as jnp
from jax.experimental import pallas as pl
from jax.experimental.pallas import tpu as pltpu

EMBED = 256
N = 8192
BM = 512
BN = 512
GI = N // BM
GJ = N // BN
NLANE = 128
NG = BN // NLANE


def _dist_argmin_body(z_ref, w2x_ref, z2_ref, w2_ref, idx_ref, rmin, rarg):
    j = pl.program_id(1)

    @pl.when(j == 0)
    def _init():
        rmin[...] = jnp.full((BM, NLANE), jnp.inf, jnp.float32)
        rarg[...] = jnp.zeros((BM, NLANE), jnp.int32)

    zb = z_ref[...]
    wb = w2x_ref[...]
    mm2 = jax.lax.dot_general(
        zb, wb, (((1,), (1,)), ((), ())),
        preferred_element_type=jnp.float32)
    d = (z2_ref[...] - mm2) + w2_ref[...]

    rm = rmin[...]
    ra = rarg[...]
    for g in range(NG):
        dg = d[:, g * NLANE:(g + 1) * NLANE]
        s = j * NG + g
        better = dg < rm
        rm = jnp.where(better, dg, rm)
        ra = jnp.where(better, jnp.int32(s), ra)
    rmin[...] = rm
    rarg[...] = ra

    @pl.when(j == GJ - 1)
    def _flush():
        col = rarg[...] * NLANE + jax.lax.broadcasted_iota(
            jnp.int32, (BM, NLANE), 1)
        gmin = jnp.min(rmin[...], axis=1, keepdims=True)
        idx = jnp.min(jnp.where(rmin[...] == gmin, col, jnp.int32(2**30)),
                      axis=1, keepdims=True)
        idx_ref[...] = idx


@functools.partial(jax.jit)
def _encode(z, W2x, z2, w2):
    return pl.pallas_call(
        _dist_argmin_body,
        grid=(GI, GJ),
        in_specs=[
            pl.BlockSpec((BM, EMBED), lambda i, j: (i, 0)),
            pl.BlockSpec((BN, EMBED), lambda i, j: (j, 0)),
            pl.BlockSpec((BM, 1), lambda i, j: (i, 0)),
            pl.BlockSpec((1, BN), lambda i, j: (0, j)),
        ],
        out_specs=pl.BlockSpec((BM, 1), lambda i, j: (i, 0)),
        out_shape=jax.ShapeDtypeStruct((N, 1), jnp.int32),
        scratch_shapes=[
            pltpu.VMEM((BM, NLANE), jnp.float32),
            pltpu.VMEM((BM, NLANE), jnp.int32),
        ],
        compiler_params=pltpu.CompilerParams(
            dimension_semantics=("parallel", "arbitrary")),
    )(z, W2x, z2, w2)


def kernel(z, W):
    z2 = jnp.sum(z ** 2, axis=1, keepdims=True)
    w2 = jnp.sum(W ** 2, axis=1)[None, :]
    idx = _encode(z, 2.0 * W, z2, w2)[:, 0]
    z_q = W[idx]
    commitment_loss = jnp.mean((jax.lax.stop_gradient(z_q) - z) ** 2)
    codebook_loss = jnp.mean((z_q - jax.lax.stop_gradient(z)) ** 2)
    vq_loss = codebook_loss + 0.25 * commitment_loss
    z_q_st = z + jax.lax.stop_gradient(z_q - z)
    return (z_q_st, vq_loss)
